# Initial kernel scaffold; baseline (speedup 1.0000x reference)
import jax
import jax.numpy as jnp
from jax.experimental import pallas as pl


def kernel(inputs, emb):
    raise NotImplementedError("write your pallas kernel here")


# speedup vs baseline: 1.2588x; 1.2588x over previous
import jax
import jax.numpy as jnp
from jax.experimental import pallas as pl
from jax.experimental.pallas import tpu as pltpu

NUM_EMB = 1024
DIM = 64
BLK = 512
COMMIT = 0.25


def _vq_block(x_ref, emb_ref, embt_ref, quant_ref, idx_ref, loss_ref):
    b = pl.program_id(0)
    s = pl.program_id(1)
    x = x_ref[0]
    e = emb_ref[...]
    et = embt_ref[...]
    xt = x.T
    xe = jax.lax.dot_general(
        xt, e, (((1,), (1,)), ((), ())),
        preferred_element_type=jnp.float32)
    x2 = jnp.sum(xt * xt, axis=1, keepdims=True)
    e2 = jnp.sum(et * et, axis=0, keepdims=True)
    sq = x2 + e2 - 2.0 * xe
    d = jnp.sqrt(jnp.maximum(sq, 0.0))
    dmin = jnp.min(d, axis=1, keepdims=True)
    lane = jax.lax.broadcasted_iota(jnp.int32, (BLK, NUM_EMB), 1)
    idx = jnp.min(jnp.where(d == dmin, lane, NUM_EMB), axis=1, keepdims=True)
    idxr = idx.T
    sub = jax.lax.broadcasted_iota(jnp.int32, (NUM_EMB, BLK), 0)
    oh = (sub == idxr).astype(jnp.float32)
    q = jnp.dot(et, oh, preferred_element_type=jnp.float32)
    quant_ref[0] = q
    idx_ref[0, 0] = idx
    diff = q - x
    partial = jnp.sum(diff * diff)

    @pl.when(jnp.logical_and(b == 0, s == 0))
    def _init():
        loss_ref[0, 0] = 0.0

    loss_ref[0, 0] += partial


def kernel(inputs, emb):
    B, C, L, H, W = inputs.shape
    N = L * H * W
    nblk = N // BLK
    x3 = inputs.reshape(B, C, N)
    embt = emb.T
    quant3, idx4, loss2 = pl.pallas_call(
        _vq_block,
        grid=(B, nblk),
        in_specs=[
            pl.BlockSpec((1, C, BLK), lambda b, s: (b, 0, s)),
            pl.BlockSpec((NUM_EMB, DIM), lambda b, s: (0, 0)),
            pl.BlockSpec((DIM, NUM_EMB), lambda b, s: (0, 0)),
        ],
        out_specs=[
            pl.BlockSpec((1, C, BLK), lambda b, s: (b, 0, s)),
            pl.BlockSpec((1, 1, BLK, 1), lambda b, s: (b, s, 0, 0)),
            pl.BlockSpec(memory_space=pltpu.SMEM),
        ],
        out_shape=[
            jax.ShapeDtypeStruct((B, C, N), jnp.float32),
            jax.ShapeDtypeStruct((B, nblk, BLK, 1), jnp.int32),
            jax.ShapeDtypeStruct((1, 1), jnp.float32),
        ],
    )(x3, emb, embt)
    quant = quant3.reshape(B, C, L, H, W)
    idx = idx4.reshape(B, L, H, W)
    loss = loss2[0, 0] * (1.25 / (B * C * N))
    return (quant, loss, idx)


# speedup vs baseline: 1.4311x; 1.1369x over previous
import jax
import jax.numpy as jnp
from jax.experimental import pallas as pl
from jax.experimental.pallas import tpu as pltpu

NUM_EMB = 1024
DIM = 64
SUB = 512
NSUB = 4
BLK = SUB * NSUB


def _vq_block(x_ref, emb_ref, embt_ref, quant_ref, idx_ref, loss_ref):
    b = pl.program_id(0)
    s = pl.program_id(1)
    e = emb_ref[...]
    et = embt_ref[...]
    e2 = jnp.sum(et * et, axis=0, keepdims=True)
    lane = jax.lax.broadcasted_iota(jnp.int32, (SUB, NUM_EMB), 1)
    sub_iota = jax.lax.broadcasted_iota(jnp.int32, (NUM_EMB, SUB), 0)
    total = jnp.zeros((), jnp.float32)
    for k in range(NSUB):
        x = x_ref[0, :, pl.ds(k * SUB, SUB)]
        xt = x.T
        xe = jax.lax.dot_general(
            xt, e, (((1,), (1,)), ((), ())),
            preferred_element_type=jnp.float32)
        x2 = jnp.sum(xt * xt, axis=1, keepdims=True)
        sq = x2 + e2 - 2.0 * xe
        d = jnp.sqrt(jnp.maximum(sq, 0.0))
        dmin = jnp.min(d, axis=1, keepdims=True)
        idx = jnp.min(jnp.where(d == dmin, lane, NUM_EMB),
                      axis=1, keepdims=True)
        idxr = idx.T
        oh = (sub_iota == idxr).astype(jnp.float32)
        q = jnp.dot(et, oh, preferred_element_type=jnp.float32)
        quant_ref[0, :, pl.ds(k * SUB, SUB)] = q
        idx_ref[0, 0, pl.ds(k * SUB, SUB)] = idx
        diff = q - x
        total = total + jnp.sum(diff * diff)

    @pl.when(jnp.logical_and(b == 0, s == 0))
    def _init():
        loss_ref[0, 0] = 0.0

    loss_ref[0, 0] += total


def kernel(inputs, emb):
    B, C, L, H, W = inputs.shape
    N = L * H * W
    nblk = N // BLK
    x3 = inputs.reshape(B, C, N)
    embt = emb.T
    quant3, idx4, loss2 = pl.pallas_call(
        _vq_block,
        grid=(B, nblk),
        in_specs=[
            pl.BlockSpec((1, C, BLK), lambda b, s: (b, 0, s)),
            pl.BlockSpec((NUM_EMB, DIM), lambda b, s: (0, 0)),
            pl.BlockSpec((DIM, NUM_EMB), lambda b, s: (0, 0)),
        ],
        out_specs=[
            pl.BlockSpec((1, C, BLK), lambda b, s: (b, 0, s)),
            pl.BlockSpec((1, 1, BLK, 1), lambda b, s: (b, s, 0, 0)),
            pl.BlockSpec(memory_space=pltpu.SMEM),
        ],
        out_shape=[
            jax.ShapeDtypeStruct((B, C, N), jnp.float32),
            jax.ShapeDtypeStruct((B, nblk, BLK, 1), jnp.int32),
            jax.ShapeDtypeStruct((1, 1), jnp.float32),
        ],
    )(x3, emb, embt)
    quant = quant3.reshape(B, C, L, H, W)
    idx = idx4.reshape(B, L, H, W)
    loss = loss2[0, 0] * (1.25 / (B * C * N))
    return (quant, loss, idx)


# speedup vs baseline: 1.4531x; 1.0154x over previous
import jax
import jax.numpy as jnp
from jax.experimental import pallas as pl
from jax.experimental.pallas import tpu as pltpu

NUM_EMB = 1024
DIM = 64
SUB = 512
NSUB = 8
BLK = SUB * NSUB


def _vq_block(x_ref, emb_ref, embt_ref, quant_ref, idx_ref, loss_ref):
    b = pl.program_id(0)
    s = pl.program_id(1)
    e = emb_ref[...]
    et = embt_ref[...]
    e2 = jnp.sum(et * et, axis=0, keepdims=True)
    lane = jax.lax.broadcasted_iota(jnp.int32, (SUB, NUM_EMB), 1)
    sub_iota = jax.lax.broadcasted_iota(jnp.int32, (NUM_EMB, SUB), 0)
    total = jnp.zeros((), jnp.float32)
    for k in range(NSUB):
        x = x_ref[0, :, pl.ds(k * SUB, SUB)]
        xt = x.T
        xe = jax.lax.dot_general(
            xt, e, (((1,), (1,)), ((), ())),
            preferred_element_type=jnp.float32)
        x2 = jnp.sum(xt * xt, axis=1, keepdims=True)
        sq = x2 + e2 - 2.0 * xe
        d = jnp.sqrt(jnp.maximum(sq, 0.0))
        dmin = jnp.min(d, axis=1, keepdims=True)
        idx = jnp.min(jnp.where(d == dmin, lane, NUM_EMB),
                      axis=1, keepdims=True)
        idxr = idx.T
        oh = (sub_iota == idxr).astype(jnp.float32)
        q = jnp.dot(et, oh, preferred_element_type=jnp.float32)
        quant_ref[0, :, pl.ds(k * SUB, SUB)] = q
        idx_ref[0, 0, pl.ds(k * SUB, SUB)] = idx
        diff = q - x
        total = total + jnp.sum(diff * diff)

    @pl.when(jnp.logical_and(b == 0, s == 0))
    def _init():
        loss_ref[0, 0] = 0.0

    loss_ref[0, 0] += total


def kernel(inputs, emb):
    B, C, L, H, W = inputs.shape
    N = L * H * W
    nblk = N // BLK
    x3 = inputs.reshape(B, C, N)
    embt = emb.T
    quant3, idx4, loss2 = pl.pallas_call(
        _vq_block,
        grid=(B, nblk),
        in_specs=[
            pl.BlockSpec((1, C, BLK), lambda b, s: (b, 0, s)),
            pl.BlockSpec((NUM_EMB, DIM), lambda b, s: (0, 0)),
            pl.BlockSpec((DIM, NUM_EMB), lambda b, s: (0, 0)),
        ],
        out_specs=[
            pl.BlockSpec((1, C, BLK), lambda b, s: (b, 0, s)),
            pl.BlockSpec((1, 1, BLK, 1), lambda b, s: (b, s, 0, 0)),
            pl.BlockSpec(memory_space=pltpu.SMEM),
        ],
        out_shape=[
            jax.ShapeDtypeStruct((B, C, N), jnp.float32),
            jax.ShapeDtypeStruct((B, nblk, BLK, 1), jnp.int32),
            jax.ShapeDtypeStruct((1, 1), jnp.float32),
        ],
    )(x3, emb, embt)
    quant = quant3.reshape(B, C, L, H, W)
    idx = idx4.reshape(B, L, H, W)
    loss = loss2[0, 0] * (1.25 / (B * C * N))
    return (quant, loss, idx)


# speedup vs baseline: 2.0030x; 1.3784x over previous
import jax
import jax.numpy as jnp
from jax.experimental import pallas as pl
from jax.experimental.pallas import tpu as pltpu

NUM_EMB = 1024
DIM = 64
SUB = 512
NSUB = 8
BLKR = SUB * NSUB


def _vq_block(x_ref, emb_ref, embt_ref, quant_ref, idx_ref, loss_ref):
    i = pl.program_id(0)
    e = emb_ref[...]
    et = embt_ref[...]
    e2 = jnp.sum(et * et, axis=0, keepdims=True)
    lane = jax.lax.broadcasted_iota(jnp.int32, (SUB, NUM_EMB), 1)
    total = jnp.zeros((), jnp.float32)
    for k in range(NSUB):
        xt = x_ref[pl.ds(k * SUB, SUB), :]
        xe = jax.lax.dot_general(
            xt, e, (((1,), (1,)), ((), ())),
            preferred_element_type=jnp.float32)
        x2 = jnp.sum(xt * xt, axis=1, keepdims=True)
        sq = x2 + e2 - 2.0 * xe
        d = jnp.sqrt(jnp.maximum(sq, 0.0))
        dmin = jnp.min(d, axis=1, keepdims=True)
        idx = jnp.min(jnp.where(d == dmin, lane, NUM_EMB),
                      axis=1, keepdims=True)
        oh = (lane == idx).astype(jnp.float32)
        q = jax.lax.dot_general(
            oh, e, (((1,), (0,)), ((), ())),
            preferred_element_type=jnp.float32)
        quant_ref[pl.ds(k * SUB, SUB), :] = q
        idx_ref[pl.ds(k * SUB, SUB), :] = idx
        diff = q - xt
        total = total + jnp.sum(diff * diff)

    @pl.when(i == 0)
    def _init():
        loss_ref[0, 0] = 0.0

    loss_ref[0, 0] += total


def kernel(inputs, emb):
    B, C, L, H, W = inputs.shape
    N = B * L * H * W
    nblk = N // BLKR
    x2d = jnp.transpose(inputs, (0, 2, 3, 4, 1)).reshape(N, C)
    embt = emb.T
    quant2, idx2, loss2 = pl.pallas_call(
        _vq_block,
        grid=(nblk,),
        in_specs=[
            pl.BlockSpec((BLKR, C), lambda i: (i, 0)),
            pl.BlockSpec((NUM_EMB, DIM), lambda i: (0, 0)),
            pl.BlockSpec((DIM, NUM_EMB), lambda i: (0, 0)),
        ],
        out_specs=[
            pl.BlockSpec((BLKR, C), lambda i: (i, 0)),
            pl.BlockSpec((BLKR, 1), lambda i: (i, 0)),
            pl.BlockSpec(memory_space=pltpu.SMEM),
        ],
        out_shape=[
            jax.ShapeDtypeStruct((N, C), jnp.float32),
            jax.ShapeDtypeStruct((N, 1), jnp.int32),
            jax.ShapeDtypeStruct((1, 1), jnp.float32),
        ],
    )(x2d, emb, embt)
    quant = jnp.transpose(quant2.reshape(B, L, H, W, C), (0, 4, 1, 2, 3))
    idx = idx2.reshape(B, L, H, W)
    loss = loss2[0, 0] * (1.25 / (N * C))
    return (quant, loss, idx)
